# Initial kernel scaffold; baseline (speedup 1.0000x reference)
import jax
import jax.numpy as jnp
from jax import lax
from jax.experimental import pallas as pl
from jax.experimental.pallas import tpu as pltpu
from jax.experimental.pallas import tpu_sc as plsc

N = 10000
E = 320000
NC = 2
NS = 16
CHUNK = 128
NCH = E // CHUNK
ROWS_PER_SUB = N // NS
D_U = 64

_vector_mesh = plsc.VectorSubcoreMesh(
    core_axis_name="c", subcore_axis_name="s", num_cores=NC, num_subcores=NS)


def _hist_body(src_hbm, out_hbm, zrow_v, ones_v, idx_v, cnt_sp):
    c = lax.axis_index("c")
    s = lax.axis_index("s")

    @pl.loop(0, ROWS_PER_SUB)
    def _zfill(i):
        zrow_v[i, :] = jnp.zeros((16,), jnp.float32)

    @pl.loop(0, CHUNK)
    def _ofill(i):
        ones_v[i, :] = jnp.ones((16,), jnp.float32)

    pltpu.sync_copy(zrow_v, cnt_sp.at[pl.ds(s * ROWS_PER_SUB, ROWS_PER_SUB)])
    plsc.subcore_barrier()

    half = NCH // NC

    @pl.loop(0, half // NS + 1)
    def _loop(t):
        r = s + t * NS

        @pl.when(r < half)
        def _():
            ch = c * half + r
            pltpu.sync_copy(src_hbm.at[pl.ds(ch * CHUNK, CHUNK)], idx_v)
            pltpu.sync_copy(ones_v, cnt_sp.at[idx_v], add=True)

    plsc.subcore_barrier()
    row0 = s * ROWS_PER_SUB
    pltpu.sync_copy(cnt_sp.at[pl.ds(row0, ROWS_PER_SUB)],
                    out_hbm.at[pl.ds(c * N + row0, ROWS_PER_SUB)])


def _degree_partials(src):
    kfn = pl.kernel(
        _hist_body,
        out_type=jax.ShapeDtypeStruct((NC * N, 16), jnp.float32),
        mesh=_vector_mesh,
        scratch_types=[
            pltpu.VMEM((ROWS_PER_SUB, 16), jnp.float32),
            pltpu.VMEM((CHUNK, 16), jnp.float32),
            pltpu.VMEM((CHUNK,), jnp.int32),
            pltpu.VMEM_SHARED((N, 16), jnp.float32),
        ],
    )
    return kfn(src)


def _seg_body(srcs_hbm, dst_hbm, u_hbm, out_hbm, z_v, idx_a, idx_b, rows_v,
              acc_sp):
    c = lax.axis_index("c")
    s = lax.axis_index("s")

    @pl.loop(0, ROWS_PER_SUB)
    def _zfill(i):
        @pl.loop(0, D_U, step=16)
        def _(j):
            z_v[i, pl.ds(j, 16)] = jnp.zeros((16,), jnp.float32)

    pltpu.sync_copy(z_v, acc_sp.at[pl.ds(s * ROWS_PER_SUB, ROWS_PER_SUB)])
    plsc.subcore_barrier()

    @pl.loop(0, NCH // NS + 1)
    def _loop(t):
        r = s + t * NS

        @pl.when(r < NCH)
        def _():
            pltpu.sync_copy(srcs_hbm.at[pl.ds(c * E + r * CHUNK, CHUNK)],
                            idx_a)
            pltpu.sync_copy(dst_hbm.at[pl.ds(r * CHUNK, CHUNK)], idx_b)
            pltpu.sync_copy(u_hbm.at[idx_a], rows_v)
            pltpu.sync_copy(rows_v, acc_sp.at[idx_b], add=True)

    plsc.subcore_barrier()
    row0 = s * ROWS_PER_SUB
    pltpu.sync_copy(acc_sp.at[pl.ds(row0, ROWS_PER_SUB)],
                    out_hbm.at[pl.ds(c * N + row0, ROWS_PER_SUB)])


def _segment_sum(srcs, dst, u):
    kfn = pl.kernel(
        _seg_body,
        out_type=jax.ShapeDtypeStruct((NC * N, D_U), jnp.float32),
        mesh=_vector_mesh,
        scratch_types=[
            pltpu.VMEM((ROWS_PER_SUB, D_U), jnp.float32),
            pltpu.VMEM((CHUNK,), jnp.int32),
            pltpu.VMEM((CHUNK,), jnp.int32),
            pltpu.VMEM((CHUNK, D_U), jnp.float32),
            pltpu.VMEM_SHARED((N, D_U), jnp.float32),
        ],
    )
    return kfn(srcs, dst, u)


def _mm_body(x_ref, w_ref, y_ref):
    y_ref[...] = jnp.dot(x_ref[0], w_ref[...],
                         preferred_element_type=jnp.float32)


def _project(xt, wcat):
    return pl.pallas_call(
        _mm_body,
        grid=(2,),
        in_specs=[
            pl.BlockSpec((1, N, 128), lambda b: (b, 0, 0)),
            pl.BlockSpec((128, 128), lambda b: (0, 0)),
        ],
        out_specs=pl.BlockSpec((N, 128), lambda b: (b, 0)),
        out_shape=jax.ShapeDtypeStruct((2 * N, 128), jnp.float32),
    )(xt, wcat)


def _scale_body(y_ref, hp_ref, u_ref):
    deg = hp_ref[0][:, 0:1] + hp_ref[1][:, 0:1]
    dinv = jnp.where(deg > 0, lax.rsqrt(deg), 0.0)
    u_ref[...] = y_ref[:, 64:] * dinv


def _scale(y, hp):
    return pl.pallas_call(
        _scale_body,
        grid=(2,),
        in_specs=[
            pl.BlockSpec((N, 128), lambda b: (b, 0)),
            pl.BlockSpec((2, N, 16), lambda b: (0, 0, 0)),
        ],
        out_specs=pl.BlockSpec((N, D_U), lambda b: (b, 0)),
        out_shape=jax.ShapeDtypeStruct((2 * N, D_U), jnp.float32),
    )(y, hp)


def _comb_body(y_ref, s_ref, hp_ref, bz_ref, bh_ref, wl_ref, bl_ref, o_ref):
    deg = hp_ref[0][:, 0:1] + hp_ref[1][:, 0:1]
    dinv = jnp.where(deg > 0, lax.rsqrt(deg), 0.0)
    a = -dinv * s_ref[...]
    z = jax.nn.sigmoid(y_ref[:, 0:32] + a[:, 0:32] + bz_ref[...])
    ht = jnp.tanh(y_ref[:, 32:64] + a[:, 32:64] + bh_ref[...])
    h = (1.0 - z) * ht
    m = jnp.sum(h, axis=0, keepdims=True) * (1.0 / N)
    o_ref[...] = jax.nn.relu(m) @ wl_ref[...] + bl_ref[...]


def _combine(y, sagg, hp, bz, bh, wlin, blin):
    return pl.pallas_call(
        _comb_body,
        grid=(2,),
        in_specs=[
            pl.BlockSpec((N, 128), lambda b: (b, 0)),
            pl.BlockSpec((N, D_U), lambda b: (b, 0)),
            pl.BlockSpec((2, N, 16), lambda b: (0, 0, 0)),
            pl.BlockSpec((1, 32), lambda b: (0, 0)),
            pl.BlockSpec((1, 32), lambda b: (0, 0)),
            pl.BlockSpec((32, 8), lambda b: (0, 0)),
            pl.BlockSpec((1, 8), lambda b: (0, 0)),
        ],
        out_specs=pl.BlockSpec((1, 8), lambda b: (b, 0)),
        out_shape=jax.ShapeDtypeStruct((2, 8), jnp.float32),
    )(y, sagg, hp, bz, bh, wlin, blin)


@jax.jit
def _run(x, edge_index, Wxz0, Wxz1, bxz, bhz, Wxh0, Wxh1, bxh, bhh, Wlin,
         blin):
    xt = x[:, -1]
    wcat = jnp.concatenate([Wxz0, Wxh0, Wxz1, Wxh1], axis=1)
    src = edge_index[0]
    dst = edge_index[1]
    srcs = (src[None, :] +
            (jnp.arange(NC, dtype=jnp.int32) * N)[:, None]).reshape(-1)

    hp = _degree_partials(src)
    y = _project(xt, wcat)
    u = _scale(y, hp.reshape(NC, N, 16))
    sagg = _segment_sum(srcs, dst, u)
    bz = (bxz + bhz).reshape(1, 32)
    bh = (bxh + bhh).reshape(1, 32)
    return _combine(y, sagg, hp.reshape(NC, N, 16), bz, bh, Wlin,
                    blin.reshape(1, 8))


def kernel(x, edge_index, Wxz0, Wxz1, bxz, Whz0, Whz1, bhz, Wxr0, Wxr1, bxr,
           Whr0, Whr1, bhr, Wxh0, Wxh1, bxh, Whh0, Whh1, bhh, Wlin, blin):
    return _run(x, edge_index, Wxz0, Wxz1, bxz, bhz, Wxh0, Wxh1, bxh, bhh,
                Wlin, blin)


# speedup vs baseline: 10.3862x; 10.3862x over previous
import jax
import jax.numpy as jnp
from jax import lax
from jax.experimental import pallas as pl
from jax.experimental.pallas import tpu as pltpu
from jax.experimental.pallas import tpu_sc as plsc

N = 10000
E = 320000
NC = 2
NS = 16
CHUNK = 128
NCH = E // CHUNK
PK_SH = 14
PK_MASK = (1 << PK_SH) - 1
RNG = 5120
AROWS = 5248
WB = RNG // NS
ZSPAN = AROWS // NS

_vector_mesh = plsc.VectorSubcoreMesh(
    core_axis_name="c", subcore_axis_name="s", num_cores=NC, num_subcores=NS)


def _hist_body(pk_hbm, zero_hbm, ones_hbm, out_hbm, ones_v, idx_p, idx_b,
               cnt_sp):
    c = lax.axis_index("c")
    s = lax.axis_index("s")
    pltpu.sync_copy(ones_hbm, ones_v)
    half = NCH // NC

    for p in range(2):
        pltpu.sync_copy(zero_hbm, cnt_sp.at[pl.ds(s * ZSPAN, ZSPAN)])
        plsc.subcore_barrier()

        @pl.loop(0, half // NS + 1)
        def _loop(t):
            r = s + t * NS

            @pl.when(r < half)
            def _():
                ch = c * half + r
                pltpu.sync_copy(pk_hbm.at[pl.ds(ch * CHUNK, CHUNK)], idx_p)
                dumpv = RNG + lax.iota(jnp.int32, 16)
                for k in range(CHUNK // 16):
                    v = idx_p[pl.ds(16 * k, 16)]
                    loc = (v & PK_MASK) - p * RNG
                    ok = (loc >= 0) & (loc < RNG)
                    idx_b[pl.ds(16 * k, 16)] = jnp.where(ok, loc, dumpv)
                pltpu.sync_copy(ones_v, cnt_sp.at[idx_b], add=True)

        plsc.subcore_barrier()
        pltpu.sync_copy(cnt_sp.at[pl.ds(s * WB, WB)],
                        out_hbm.at[(c * 2 + p) * NS + s])
        plsc.subcore_barrier()


def _degree_partials(packed, zero128, ones128):
    kfn = pl.kernel(
        _hist_body,
        out_type=jax.ShapeDtypeStruct((NC * 2 * NS, WB, 128), jnp.float32),
        mesh=_vector_mesh,
        scratch_types=[
            pltpu.VMEM((CHUNK, 128), jnp.float32),
            pltpu.VMEM((CHUNK,), jnp.int32),
            pltpu.VMEM((CHUNK,), jnp.int32),
            pltpu.VMEM_SHARED((AROWS, 128), jnp.float32),
        ],
    )
    return kfn(packed, zero128, ones128)


def _seg_body(pk_hbm, zero_hbm, u_hbm, out_hbm, idx_p, idx_a, idx_b, rows_v,
              acc_sp):
    c = lax.axis_index("c")
    s = lax.axis_index("s")

    for p in range(2):
        pltpu.sync_copy(zero_hbm, acc_sp.at[pl.ds(s * ZSPAN, ZSPAN)])
        plsc.subcore_barrier()

        @pl.loop(0, NCH // NS + 1)
        def _loop(t):
            r = s + t * NS

            @pl.when(r < NCH)
            def _():
                pltpu.sync_copy(pk_hbm.at[pl.ds(r * CHUNK, CHUNK)], idx_p)
                dumpv = RNG + lax.iota(jnp.int32, 16)
                for k in range(CHUNK // 16):
                    v = idx_p[pl.ds(16 * k, 16)]
                    idx_a[pl.ds(16 * k, 16)] = (v & PK_MASK) + c * N
                    loc = (v >> PK_SH) - p * RNG
                    ok = (loc >= 0) & (loc < RNG)
                    idx_b[pl.ds(16 * k, 16)] = jnp.where(ok, loc, dumpv)
                pltpu.sync_copy(u_hbm.at[idx_a], rows_v)
                pltpu.sync_copy(rows_v, acc_sp.at[idx_b], add=True)

        plsc.subcore_barrier()
        pltpu.sync_copy(acc_sp.at[pl.ds(s * WB, WB)],
                        out_hbm.at[(c * 2 + p) * NS + s])
        plsc.subcore_barrier()


def _segment_sum(packed, zero128, u):
    kfn = pl.kernel(
        _seg_body,
        out_type=jax.ShapeDtypeStruct((NC * 2 * NS, WB, 128), jnp.float32),
        mesh=_vector_mesh,
        scratch_types=[
            pltpu.VMEM((CHUNK,), jnp.int32),
            pltpu.VMEM((CHUNK,), jnp.int32),
            pltpu.VMEM((CHUNK,), jnp.int32),
            pltpu.VMEM((CHUNK, 128), jnp.float32),
            pltpu.VMEM_SHARED((AROWS, 128), jnp.float32),
        ],
    )
    return kfn(packed, zero128, u)


def _mm_body(x_ref, w_ref, y_ref):
    y_ref[...] = jnp.dot(x_ref[0], w_ref[...],
                         preferred_element_type=jnp.float32)


def _project(xt, wcat):
    return pl.pallas_call(
        _mm_body,
        grid=(2,),
        in_specs=[
            pl.BlockSpec((1, N, 128), lambda b: (b, 0, 0)),
            pl.BlockSpec((128, 128), lambda b: (0, 0)),
        ],
        out_specs=pl.BlockSpec((N, 128), lambda b: (b, 0)),
        out_shape=jax.ShapeDtypeStruct((2 * N, 128), jnp.float32),
    )(xt, wcat)


def _scale_body(y_ref, hp_ref, u_ref):
    deg = hp_ref[0][:, 0:1] + hp_ref[1][:, 0:1]
    dinv = jnp.where(deg > 0, lax.rsqrt(deg), 0.0)
    u_ref[...] = y_ref[...] * dinv


def _scale(y, hp):
    return pl.pallas_call(
        _scale_body,
        grid=(2,),
        in_specs=[
            pl.BlockSpec((N, 128), lambda b: (b, 0)),
            pl.BlockSpec((2, N, 16), lambda b: (0, 0, 0)),
        ],
        out_specs=pl.BlockSpec((N, 128), lambda b: (b, 0)),
        out_shape=jax.ShapeDtypeStruct((2 * N, 128), jnp.float32),
    )(y, hp)


def _comb_body(y_ref, s_ref, hp_ref, bz_ref, bh_ref, wl_ref, bl_ref, o_ref):
    deg = hp_ref[0][:, 0:1] + hp_ref[1][:, 0:1]
    dinv = jnp.where(deg > 0, lax.rsqrt(deg), 0.0)
    a = -dinv * s_ref[:, 64:128]
    z = jax.nn.sigmoid(y_ref[:, 0:32] + a[:, 0:32] + bz_ref[...])
    ht = jnp.tanh(y_ref[:, 32:64] + a[:, 32:64] + bh_ref[...])
    h = (1.0 - z) * ht
    m = jnp.sum(h, axis=0, keepdims=True) * (1.0 / N)
    b = pl.program_id(0)
    o_ref[pl.ds(b, 1), :] = jax.nn.relu(m) @ wl_ref[...] + bl_ref[...]


def _combine(y, sagg, hp, bz, bh, wlin, blin):
    return pl.pallas_call(
        _comb_body,
        grid=(2,),
        in_specs=[
            pl.BlockSpec((N, 128), lambda b: (b, 0)),
            pl.BlockSpec((N, 128), lambda b: (b, 0)),
            pl.BlockSpec((2, N, 16), lambda b: (0, 0, 0)),
            pl.BlockSpec((1, 32), lambda b: (0, 0)),
            pl.BlockSpec((1, 32), lambda b: (0, 0)),
            pl.BlockSpec((32, 8), lambda b: (0, 0)),
            pl.BlockSpec((1, 8), lambda b: (0, 0)),
        ],
        out_specs=pl.BlockSpec((2, 8), lambda b: (0, 0)),
        out_shape=jax.ShapeDtypeStruct((2, 8), jnp.float32),
    )(y, sagg, hp, bz, bh, wlin, blin)


def _unsplit(o):
    o = o.reshape(NC, 2, RNG, 128)
    return jnp.concatenate([o[:, 0], o[:, 1, :N - RNG]], axis=1)


@jax.jit
def _run(x, edge_index, Wxz0, Wxz1, bxz, bhz, Wxh0, Wxh1, bxh, bhh, Wlin,
         blin):
    xt = x[:, -1]
    wcat = jnp.concatenate([Wxz0, Wxh0, Wxz1, Wxh1], axis=1)
    src = edge_index[0]
    dst = edge_index[1]
    packed = (dst << PK_SH) | src
    zero128 = jnp.zeros((ZSPAN, 128), jnp.float32)
    ones128 = jnp.ones((CHUNK, 128), jnp.float32)

    hpo = _degree_partials(packed, zero128, ones128)
    hp = _unsplit(hpo)[:, :, 0:16]
    y = _project(xt, wcat)
    u = _scale(y, hp)
    sagg = _unsplit(_segment_sum(packed, zero128, u)).reshape(NC * N, 128)
    bz = (bxz + bhz).reshape(1, 32)
    bh = (bxh + bhh).reshape(1, 32)
    return _combine(y, sagg, hp, bz, bh, Wlin, blin.reshape(1, 8))


def kernel(x, edge_index, Wxz0, Wxz1, bxz, Whz0, Whz1, bhz, Wxr0, Wxr1, bxr,
           Whr0, Whr1, bhr, Wxh0, Wxh1, bxh, Whh0, Whh1, bhh, Wlin, blin):
    return _run(x, edge_index, Wxz0, Wxz1, bxz, bhz, Wxh0, Wxh1, bxh, bhh,
                Wlin, blin)


# speedup vs baseline: 13.9375x; 1.3419x over previous
import jax
import jax.numpy as jnp
from jax import lax
from jax.experimental import pallas as pl
from jax.experimental.pallas import tpu as pltpu
from jax.experimental.pallas import tpu_sc as plsc

N = 10000
E = 320000
NC = 2
NS = 16
CHUNK = 128
NCH = E // CHUNK
PK_SH = 14
PK_MASK = (1 << PK_SH) - 1
RNG = 5120
AROWS = 5248
WB = RNG // NS
ZSPAN = AROWS // NS

_vector_mesh = plsc.VectorSubcoreMesh(
    core_axis_name="c", subcore_axis_name="s", num_cores=NC, num_subcores=NS)


def _hist_body(pk_hbm, zero_hbm, ones_hbm, out_hbm, ones_v, idx_p0, idx_p1,
               idx_b0, idx_b1, cnt_sp, sem0, sem1):
    c = lax.axis_index("c")
    s = lax.axis_index("s")
    pltpu.sync_copy(ones_hbm, ones_v)
    half = NCH // NC

    def stage(p, t, idx_p, idx_b, sem):
        r = s + t * NS

        @pl.when(r < half)
        def _():
            ch = c * half + r
            pltpu.sync_copy(pk_hbm.at[pl.ds(ch * CHUNK, CHUNK)], idx_p)
            dumpv = RNG + lax.iota(jnp.int32, 16)
            for k in range(CHUNK // 16):
                v = idx_p[pl.ds(16 * k, 16)]
                loc = (v & PK_MASK) - p * RNG
                ok = (loc >= 0) & (loc < RNG)
                idx_b[pl.ds(16 * k, 16)] = jnp.where(ok, loc, dumpv)
            pltpu.async_copy(ones_v, cnt_sp.at[idx_b], sem, add=True)

        return r

    for p in range(2):
        pltpu.sync_copy(zero_hbm, cnt_sp.at[pl.ds(s * ZSPAN, ZSPAN)])
        plsc.subcore_barrier()

        @pl.loop(0, (half // NS + 2) // 2)
        def _loop(j):
            ra = stage(p, 2 * j, idx_p0, idx_b0, sem0)
            rb = stage(p, 2 * j + 1, idx_p1, idx_b1, sem1)

            @pl.when(ra < half)
            def _():
                pltpu.make_async_copy(ones_v, cnt_sp.at[idx_b0], sem0).wait()

            @pl.when(rb < half)
            def _():
                pltpu.make_async_copy(ones_v, cnt_sp.at[idx_b1], sem1).wait()

        plsc.subcore_barrier()
        pltpu.sync_copy(cnt_sp.at[pl.ds(s * WB, WB)],
                        out_hbm.at[(c * 2 + p) * NS + s])
        plsc.subcore_barrier()


def _degree_partials(packed, zero128, ones128):
    kfn = pl.kernel(
        _hist_body,
        out_type=jax.ShapeDtypeStruct((NC * 2 * NS, WB, 128), jnp.float32),
        mesh=_vector_mesh,
        scratch_types=[
            pltpu.VMEM((CHUNK, 128), jnp.float32),
            pltpu.VMEM((CHUNK,), jnp.int32),
            pltpu.VMEM((CHUNK,), jnp.int32),
            pltpu.VMEM((CHUNK,), jnp.int32),
            pltpu.VMEM((CHUNK,), jnp.int32),
            pltpu.VMEM_SHARED((AROWS, 128), jnp.float32),
            pltpu.SemaphoreType.DMA,
            pltpu.SemaphoreType.DMA,
        ],
    )
    return kfn(packed, zero128, ones128)


def _seg_body(pk_hbm, zero_hbm, u_hbm, out_hbm, idx_p0, idx_p1, idx_a0,
              idx_a1, idx_b0, idx_b1, rows0, rows1, acc_sp, semg0, semg1,
              sems0, sems1):
    c = lax.axis_index("c")
    s = lax.axis_index("s")

    def load_unpack_gather(p, t, idx_p, idx_a, idx_b, rows_v, semg):
        r = s + t * NS

        @pl.when(r < NCH)
        def _():
            pltpu.sync_copy(pk_hbm.at[pl.ds(r * CHUNK, CHUNK)], idx_p)
            dumpv = RNG + lax.iota(jnp.int32, 16)
            for k in range(CHUNK // 16):
                v = idx_p[pl.ds(16 * k, 16)]
                idx_a[pl.ds(16 * k, 16)] = (v & PK_MASK) + c * N
                loc = (v >> PK_SH) - p * RNG
                ok = (loc >= 0) & (loc < RNG)
                idx_b[pl.ds(16 * k, 16)] = jnp.where(ok, loc, dumpv)
            pltpu.async_copy(u_hbm.at[idx_a], rows_v, semg)

        return r

    def scatter(r, idx_a, idx_b, rows_v, semg, sems):
        @pl.when(r < NCH)
        def _():
            pltpu.make_async_copy(u_hbm.at[idx_a], rows_v, semg).wait()
            pltpu.async_copy(rows_v, acc_sp.at[idx_b], sems, add=True)

    def scatter_wait(r, idx_b, rows_v, sems):
        @pl.when(r < NCH)
        def _():
            pltpu.make_async_copy(rows_v, acc_sp.at[idx_b], sems).wait()

    for p in range(2):
        pltpu.sync_copy(zero_hbm, acc_sp.at[pl.ds(s * ZSPAN, ZSPAN)])
        plsc.subcore_barrier()

        @pl.loop(0, (NCH // NS + 2) // 2)
        def _loop(j):
            ra = load_unpack_gather(p, 2 * j, idx_p0, idx_a0, idx_b0, rows0,
                                    semg0)
            rb = load_unpack_gather(p, 2 * j + 1, idx_p1, idx_a1, idx_b1,
                                    rows1, semg1)
            scatter(ra, idx_a0, idx_b0, rows0, semg0, sems0)
            scatter(rb, idx_a1, idx_b1, rows1, semg1, sems1)
            scatter_wait(ra, idx_b0, rows0, sems0)
            scatter_wait(rb, idx_b1, rows1, sems1)

        plsc.subcore_barrier()
        pltpu.sync_copy(acc_sp.at[pl.ds(s * WB, WB)],
                        out_hbm.at[(c * 2 + p) * NS + s])
        plsc.subcore_barrier()


def _segment_sum(packed, zero128, u):
    kfn = pl.kernel(
        _seg_body,
        out_type=jax.ShapeDtypeStruct((NC * 2 * NS, WB, 128), jnp.float32),
        mesh=_vector_mesh,
        scratch_types=[
            pltpu.VMEM((CHUNK,), jnp.int32),
            pltpu.VMEM((CHUNK,), jnp.int32),
            pltpu.VMEM((CHUNK,), jnp.int32),
            pltpu.VMEM((CHUNK,), jnp.int32),
            pltpu.VMEM((CHUNK,), jnp.int32),
            pltpu.VMEM((CHUNK,), jnp.int32),
            pltpu.VMEM((CHUNK, 128), jnp.float32),
            pltpu.VMEM((CHUNK, 128), jnp.float32),
            pltpu.VMEM_SHARED((AROWS, 128), jnp.float32),
            pltpu.SemaphoreType.DMA,
            pltpu.SemaphoreType.DMA,
            pltpu.SemaphoreType.DMA,
            pltpu.SemaphoreType.DMA,
        ],
    )
    return kfn(packed, zero128, u)


def _mm_body(x_ref, w_ref, y_ref):
    y_ref[...] = jnp.dot(x_ref[0], w_ref[...],
                         preferred_element_type=jnp.float32)


def _project(xt, wcat):
    return pl.pallas_call(
        _mm_body,
        grid=(2,),
        in_specs=[
            pl.BlockSpec((1, N, 128), lambda b: (b, 0, 0)),
            pl.BlockSpec((128, 128), lambda b: (0, 0)),
        ],
        out_specs=pl.BlockSpec((N, 128), lambda b: (b, 0)),
        out_shape=jax.ShapeDtypeStruct((2 * N, 128), jnp.float32),
    )(xt, wcat)


def _scale_body(y_ref, hp_ref, u_ref):
    deg = hp_ref[0][:, 0:1] + hp_ref[1][:, 0:1]
    dinv = jnp.where(deg > 0, lax.rsqrt(deg), 0.0)
    u_ref[...] = y_ref[...] * dinv


def _scale(y, hp):
    return pl.pallas_call(
        _scale_body,
        grid=(2,),
        in_specs=[
            pl.BlockSpec((N, 128), lambda b: (b, 0)),
            pl.BlockSpec((2, N, 16), lambda b: (0, 0, 0)),
        ],
        out_specs=pl.BlockSpec((N, 128), lambda b: (b, 0)),
        out_shape=jax.ShapeDtypeStruct((2 * N, 128), jnp.float32),
    )(y, hp)


def _comb_body(y_ref, s_ref, hp_ref, bz_ref, bh_ref, wl_ref, bl_ref, o_ref):
    deg = hp_ref[0][:, 0:1] + hp_ref[1][:, 0:1]
    dinv = jnp.where(deg > 0, lax.rsqrt(deg), 0.0)
    a = -dinv * s_ref[:, 64:128]
    z = jax.nn.sigmoid(y_ref[:, 0:32] + a[:, 0:32] + bz_ref[...])
    ht = jnp.tanh(y_ref[:, 32:64] + a[:, 32:64] + bh_ref[...])
    h = (1.0 - z) * ht
    m = jnp.sum(h, axis=0, keepdims=True) * (1.0 / N)
    b = pl.program_id(0)
    o_ref[pl.ds(b, 1), :] = jax.nn.relu(m) @ wl_ref[...] + bl_ref[...]


def _combine(y, sagg, hp, bz, bh, wlin, blin):
    return pl.pallas_call(
        _comb_body,
        grid=(2,),
        in_specs=[
            pl.BlockSpec((N, 128), lambda b: (b, 0)),
            pl.BlockSpec((N, 128), lambda b: (b, 0)),
            pl.BlockSpec((2, N, 16), lambda b: (0, 0, 0)),
            pl.BlockSpec((1, 32), lambda b: (0, 0)),
            pl.BlockSpec((1, 32), lambda b: (0, 0)),
            pl.BlockSpec((32, 8), lambda b: (0, 0)),
            pl.BlockSpec((1, 8), lambda b: (0, 0)),
        ],
        out_specs=pl.BlockSpec((2, 8), lambda b: (0, 0)),
        out_shape=jax.ShapeDtypeStruct((2, 8), jnp.float32),
    )(y, sagg, hp, bz, bh, wlin, blin)


def _unsplit(o):
    o = o.reshape(NC, 2, RNG, 128)
    return jnp.concatenate([o[:, 0], o[:, 1, :N - RNG]], axis=1)


@jax.jit
def _run(x, edge_index, Wxz0, Wxz1, bxz, bhz, Wxh0, Wxh1, bxh, bhh, Wlin,
         blin):
    xt = x[:, -1]
    wcat = jnp.concatenate([Wxz0, Wxh0, Wxz1, Wxh1], axis=1)
    src = edge_index[0]
    dst = edge_index[1]
    packed = (dst << PK_SH) | src
    zero128 = jnp.zeros((ZSPAN, 128), jnp.float32)
    ones128 = jnp.ones((CHUNK, 128), jnp.float32)

    hpo = _degree_partials(packed, zero128, ones128)
    hp = _unsplit(hpo)[:, :, 0:16]
    y = _project(xt, wcat)
    u = _scale(y, hp)
    sagg = _unsplit(_segment_sum(packed, zero128, u)).reshape(NC * N, 128)
    bz = (bxz + bhz).reshape(1, 32)
    bh = (bxh + bhh).reshape(1, 32)
    return _combine(y, sagg, hp, bz, bh, Wlin, blin.reshape(1, 8))


def kernel(x, edge_index, Wxz0, Wxz1, bxz, Whz0, Whz1, bhz, Wxr0, Wxr1, bxr,
           Whr0, Whr1, bhr, Wxh0, Wxh1, bxh, Whh0, Whh1, bhh, Wlin, blin):
    return _run(x, edge_index, Wxz0, Wxz1, bxz, bhz, Wxh0, Wxh1, bxh, bhh,
                Wlin, blin)


# speedup vs baseline: 19.5778x; 1.4047x over previous
import jax
import jax.numpy as jnp
from jax import lax
from jax.experimental import pallas as pl
from jax.experimental.pallas import tpu as pltpu
from jax.experimental.pallas import tpu_sc as plsc

N = 10000
E = 320000
NC = 2
NS = 16
CHUNK = 128
NCH = E // CHUNK
PK_SH = 14
PK_MASK = (1 << PK_SH) - 1
RNG = 5120
AROWS = 5248
WB = RNG // NS
ZSPAN = AROWS // NS
PROWS = 5008
PSPAN = PROWS // NS

_vector_mesh = plsc.VectorSubcoreMesh(
    core_axis_name="c", subcore_axis_name="s", num_cores=NC, num_subcores=NS)


def _hist_body(pk_hbm, zero_hbm, ones_hbm, out_hbm, ones_v, idx_p0, idx_p1,
               idx_b0, idx_b1, cnt_sp, sem0, sem1):
    c = lax.axis_index("c")
    s = lax.axis_index("s")
    pltpu.sync_copy(ones_hbm, ones_v)
    half = NCH // NC

    def stage(p, t, idx_p, idx_b, sem):
        r = s + t * NS

        @pl.when(r < half)
        def _():
            ch = c * half + r
            pltpu.sync_copy(pk_hbm.at[pl.ds(ch * CHUNK, CHUNK)], idx_p)
            dumpv = RNG + lax.iota(jnp.int32, 16)
            for k in range(CHUNK // 16):
                v = idx_p[pl.ds(16 * k, 16)]
                loc = (v & PK_MASK) - p * RNG
                ok = (loc >= 0) & (loc < RNG)
                idx_b[pl.ds(16 * k, 16)] = jnp.where(ok, loc, dumpv)
            pltpu.async_copy(ones_v, cnt_sp.at[idx_b], sem, add=True)

        return r

    for p in range(2):
        pltpu.sync_copy(zero_hbm, cnt_sp.at[pl.ds(s * ZSPAN, ZSPAN)])
        plsc.subcore_barrier()

        @pl.loop(0, (half // NS + 2) // 2)
        def _loop(j):
            ra = stage(p, 2 * j, idx_p0, idx_b0, sem0)
            rb = stage(p, 2 * j + 1, idx_p1, idx_b1, sem1)

            @pl.when(ra < half)
            def _():
                pltpu.make_async_copy(ones_v, cnt_sp.at[idx_b0], sem0).wait()

            @pl.when(rb < half)
            def _():
                pltpu.make_async_copy(ones_v, cnt_sp.at[idx_b1], sem1).wait()

        plsc.subcore_barrier()
        pltpu.sync_copy(cnt_sp.at[pl.ds(s * WB, WB)],
                        out_hbm.at[(c * 2 + p) * NS + s])
        plsc.subcore_barrier()


def _degree_partials(packed, zero128, ones128):
    kfn = pl.kernel(
        _hist_body,
        out_type=jax.ShapeDtypeStruct((NC * 2 * NS, WB, 128), jnp.float32),
        mesh=_vector_mesh,
        scratch_types=[
            pltpu.VMEM((CHUNK, 128), jnp.float32),
            pltpu.VMEM((CHUNK,), jnp.int32),
            pltpu.VMEM((CHUNK,), jnp.int32),
            pltpu.VMEM((CHUNK,), jnp.int32),
            pltpu.VMEM((CHUNK,), jnp.int32),
            pltpu.VMEM_SHARED((AROWS, 128), jnp.float32),
            pltpu.SemaphoreType.DMA,
            pltpu.SemaphoreType.DMA,
        ],
    )
    return kfn(packed, zero128, ones128)


def _seg_body(pk_hbm, zero_hbm, u_hbm, out_hbm, idx_p0, idx_p1, idx_a0,
              idx_a1, idx_b0, idx_b1, rows0, rows1, acc_sp, semg0, semg1,
              sems0, sems1):
    c = lax.axis_index("c")
    s = lax.axis_index("s")

    def load_unpack_gather(t, idx_p, idx_a, idx_b, rows_v, semg):
        r = s + t * NS

        @pl.when(r < NCH)
        def _():
            pltpu.sync_copy(pk_hbm.at[pl.ds(r * CHUNK, CHUNK)], idx_p)
            for k in range(CHUNK // 16):
                v = idx_p[pl.ds(16 * k, 16)]
                d = v >> PK_SH
                idx_a[pl.ds(16 * k, 16)] = ((v & PK_MASK) + c * N
                                            + (d & 1) * (2 * N))
                idx_b[pl.ds(16 * k, 16)] = d >> 1
            pltpu.async_copy(u_hbm.at[idx_a], rows_v, semg)

        return r

    def scatter(r, idx_a, idx_b, rows_v, semg, sems):
        @pl.when(r < NCH)
        def _():
            pltpu.make_async_copy(u_hbm.at[idx_a], rows_v, semg).wait()
            pltpu.async_copy(rows_v, acc_sp.at[idx_b], sems, add=True)

    def scatter_wait(r, idx_b, rows_v, sems):
        @pl.when(r < NCH)
        def _():
            pltpu.make_async_copy(rows_v, acc_sp.at[idx_b], sems).wait()

    pltpu.sync_copy(zero_hbm, acc_sp.at[pl.ds(s * PSPAN, PSPAN)])
    plsc.subcore_barrier()

    @pl.loop(0, (NCH // NS + 2) // 2)
    def _loop(j):
        ra = load_unpack_gather(2 * j, idx_p0, idx_a0, idx_b0, rows0, semg0)
        rb = load_unpack_gather(2 * j + 1, idx_p1, idx_a1, idx_b1, rows1,
                                semg1)
        scatter(ra, idx_a0, idx_b0, rows0, semg0, sems0)
        scatter(rb, idx_a1, idx_b1, rows1, semg1, sems1)
        scatter_wait(ra, idx_b0, rows0, sems0)
        scatter_wait(rb, idx_b1, rows1, sems1)

    plsc.subcore_barrier()
    pltpu.sync_copy(acc_sp.at[pl.ds(s * PSPAN, PSPAN)],
                    out_hbm.at[c * NS + s])


def _segment_sum(packed, zerop, u):
    kfn = pl.kernel(
        _seg_body,
        out_type=jax.ShapeDtypeStruct((NC * NS, PSPAN, 128), jnp.float32),
        mesh=_vector_mesh,
        scratch_types=[
            pltpu.VMEM((CHUNK,), jnp.int32),
            pltpu.VMEM((CHUNK,), jnp.int32),
            pltpu.VMEM((CHUNK,), jnp.int32),
            pltpu.VMEM((CHUNK,), jnp.int32),
            pltpu.VMEM((CHUNK,), jnp.int32),
            pltpu.VMEM((CHUNK,), jnp.int32),
            pltpu.VMEM((CHUNK, 128), jnp.float32),
            pltpu.VMEM((CHUNK, 128), jnp.float32),
            pltpu.VMEM_SHARED((PROWS, 128), jnp.float32),
            pltpu.SemaphoreType.DMA,
            pltpu.SemaphoreType.DMA,
            pltpu.SemaphoreType.DMA,
            pltpu.SemaphoreType.DMA,
        ],
    )
    return kfn(packed, zerop, u)


def _mm_body(x_ref, w_ref, y_ref):
    y_ref[...] = jnp.dot(x_ref[0], w_ref[...],
                         preferred_element_type=jnp.float32)


def _project(xt, wcat):
    return pl.pallas_call(
        _mm_body,
        grid=(2,),
        in_specs=[
            pl.BlockSpec((1, N, 128), lambda b: (b, 0, 0)),
            pl.BlockSpec((128, 128), lambda b: (0, 0)),
        ],
        out_specs=pl.BlockSpec((N, 128), lambda b: (b, 0)),
        out_shape=jax.ShapeDtypeStruct((2 * N, 128), jnp.float32),
    )(xt, wcat)


def _scale_body(y_ref, hp_ref, u_ref):
    arr = pl.program_id(0) // 2
    deg = hp_ref[0][:, 0:1] + hp_ref[1][:, 0:1]
    dinv = jnp.where(deg > 0, lax.rsqrt(deg), 0.0)
    u64 = y_ref[:, 64:128] * dinv
    u_ref[:, 0:64] = jnp.where(arr == 0, u64, 0.0)
    u_ref[:, 64:128] = jnp.where(arr == 0, 0.0, u64)


def _scale(y, hp):
    return pl.pallas_call(
        _scale_body,
        grid=(4,),
        in_specs=[
            pl.BlockSpec((N, 128), lambda g: (g % 2, 0)),
            pl.BlockSpec((2, N, 16), lambda g: (0, 0, 0)),
        ],
        out_specs=pl.BlockSpec((N, 128), lambda g: (g, 0)),
        out_shape=jax.ShapeDtypeStruct((4 * N, 128), jnp.float32),
    )(y, hp)


def _comb_body(y_ref, s_ref, hp_ref, bz_ref, bh_ref, wl_ref, bl_ref, o_ref):
    deg = hp_ref[0][:, 0:1] + hp_ref[1][:, 0:1]
    dinv = jnp.where(deg > 0, lax.rsqrt(deg), 0.0)
    a = -dinv * s_ref[...]
    z = jax.nn.sigmoid(y_ref[:, 0:32] + a[:, 0:32] + bz_ref[...])
    ht = jnp.tanh(y_ref[:, 32:64] + a[:, 32:64] + bh_ref[...])
    h = (1.0 - z) * ht
    m = jnp.sum(h, axis=0, keepdims=True) * (1.0 / N)
    b = pl.program_id(0)
    o_ref[pl.ds(b, 1), :] = jax.nn.relu(m) @ wl_ref[...] + bl_ref[...]


def _combine(y, sagg, hp, bz, bh, wlin, blin):
    return pl.pallas_call(
        _comb_body,
        grid=(2,),
        in_specs=[
            pl.BlockSpec((N, 128), lambda b: (b, 0)),
            pl.BlockSpec((N, 64), lambda b: (b, 0)),
            pl.BlockSpec((2, N, 16), lambda b: (0, 0, 0)),
            pl.BlockSpec((1, 32), lambda b: (0, 0)),
            pl.BlockSpec((1, 32), lambda b: (0, 0)),
            pl.BlockSpec((32, 8), lambda b: (0, 0)),
            pl.BlockSpec((1, 8), lambda b: (0, 0)),
        ],
        out_specs=pl.BlockSpec((2, 8), lambda b: (0, 0)),
        out_shape=jax.ShapeDtypeStruct((2, 8), jnp.float32),
    )(y, sagg, hp, bz, bh, wlin, blin)


def _unsplit(o):
    o = o.reshape(NC, 2, RNG, 128)
    return jnp.concatenate([o[:, 0], o[:, 1, :N - RNG]], axis=1)


def _unpair(o):
    o = o.reshape(NC, PROWS, 128)[:, :N // 2]
    s = jnp.stack([o[:, :, 0:64], o[:, :, 64:128]], axis=2)
    return s.reshape(NC * N, 64)


@jax.jit
def _run(x, edge_index, Wxz0, Wxz1, bxz, bhz, Wxh0, Wxh1, bxh, bhh, Wlin,
         blin):
    xt = x[:, -1]
    wcat = jnp.concatenate([Wxz0, Wxh0, Wxz1, Wxh1], axis=1)
    src = edge_index[0]
    dst = edge_index[1]
    packed = (dst << PK_SH) | src
    zero128 = jnp.zeros((ZSPAN, 128), jnp.float32)
    zerop = jnp.zeros((PSPAN, 128), jnp.float32)
    ones128 = jnp.ones((CHUNK, 128), jnp.float32)

    hpo = _degree_partials(packed, zero128, ones128)
    hp = _unsplit(hpo)[:, :, 0:16]
    y = _project(xt, wcat)
    u = _scale(y, hp)
    sagg = _unpair(_segment_sum(packed, zerop, u))
    bz = (bxz + bhz).reshape(1, 32)
    bh = (bxh + bhh).reshape(1, 32)
    return _combine(y, sagg, hp, bz, bh, Wlin, blin.reshape(1, 8))


def kernel(x, edge_index, Wxz0, Wxz1, bxz, Whz0, Whz1, bhz, Wxr0, Wxr1, bxr,
           Whr0, Whr1, bhr, Wxh0, Wxh1, bxh, Whh0, Whh1, bhh, Wlin, blin):
    return _run(x, edge_index, Wxz0, Wxz1, bxz, bhz, Wxh0, Wxh1, bxh, bhh,
                Wlin, blin)


# speedup vs baseline: 22.9263x; 1.1710x over previous
import jax
import jax.numpy as jnp
from jax import lax
from jax.experimental import pallas as pl
from jax.experimental.pallas import tpu as pltpu
from jax.experimental.pallas import tpu_sc as plsc

N = 10000
E = 320000
NC = 2
NS = 16
CHUNK = 128
NCH = E // CHUNK
PK_SH = 14
PK_MASK = (1 << PK_SH) - 1
RNG = 5120
AROWS = 5248
WB = RNG // NS
ZSPAN = AROWS // NS
PROWS = 5008
PSPAN = PROWS // NS

_vector_mesh = plsc.VectorSubcoreMesh(
    core_axis_name="c", subcore_axis_name="s", num_cores=NC, num_subcores=NS)


def _hist_body(pk_hbm, zero_hbm, ones_hbm, out_hbm, ones_v, idx_p0, idx_p1,
               idx_b0, idx_b1, cnt_sp, sem0, sem1):
    c = lax.axis_index("c")
    s = lax.axis_index("s")
    pltpu.sync_copy(ones_hbm, ones_v)
    half = NCH // NC

    def stage(p, t, idx_p, idx_b, sem):
        r = s + t * NS

        @pl.when(r < half)
        def _():
            ch = c * half + r
            pltpu.sync_copy(pk_hbm.at[pl.ds(ch * CHUNK, CHUNK)], idx_p)
            dumpv = RNG + lax.iota(jnp.int32, 16)
            for k in range(CHUNK // 16):
                v = idx_p[pl.ds(16 * k, 16)]
                loc = (v & PK_MASK) - p * RNG
                ok = (loc >= 0) & (loc < RNG)
                idx_b[pl.ds(16 * k, 16)] = jnp.where(ok, loc, dumpv)
            pltpu.async_copy(ones_v, cnt_sp.at[idx_b], sem, add=True)

        return r

    for p in range(2):
        pltpu.sync_copy(zero_hbm, cnt_sp.at[pl.ds(s * ZSPAN, ZSPAN)])
        plsc.subcore_barrier()

        @pl.loop(0, (half // NS + 2) // 2)
        def _loop(j):
            ra = stage(p, 2 * j, idx_p0, idx_b0, sem0)
            rb = stage(p, 2 * j + 1, idx_p1, idx_b1, sem1)

            @pl.when(ra < half)
            def _():
                pltpu.make_async_copy(ones_v, cnt_sp.at[idx_b0], sem0).wait()

            @pl.when(rb < half)
            def _():
                pltpu.make_async_copy(ones_v, cnt_sp.at[idx_b1], sem1).wait()

        plsc.subcore_barrier()
        pltpu.sync_copy(cnt_sp.at[pl.ds(s * WB, WB)],
                        out_hbm.at[(c * 2 + p) * NS + s])
        plsc.subcore_barrier()


def _degree_partials(packed, zero128, ones128):
    kfn = pl.kernel(
        _hist_body,
        out_type=jax.ShapeDtypeStruct((NC * 2 * NS, WB, 128), jnp.float32),
        mesh=_vector_mesh,
        scratch_types=[
            pltpu.VMEM((CHUNK, 128), jnp.float32),
            pltpu.VMEM((CHUNK,), jnp.int32),
            pltpu.VMEM((CHUNK,), jnp.int32),
            pltpu.VMEM((CHUNK,), jnp.int32),
            pltpu.VMEM((CHUNK,), jnp.int32),
            pltpu.VMEM_SHARED((AROWS, 128), jnp.float32),
            pltpu.SemaphoreType.DMA,
            pltpu.SemaphoreType.DMA,
        ],
    )
    return kfn(packed, zero128, ones128)


def _seg_body(pk_hbm, zero_hbm, u_hbm, out_hbm, ip0, ip1, ip2, ip3, ia0,
              ia1, ia2, ia3, ib0, ib1, ib2, ib3, rw0, rw1, rw2, rw3, acc_sp,
              sg0, sg1, sg2, sg3, ss0, ss1, ss2, ss3):
    idx_ps = [ip0, ip1, ip2, ip3]
    idx_as = [ia0, ia1, ia2, ia3]
    idx_bs = [ib0, ib1, ib2, ib3]
    rowss = [rw0, rw1, rw2, rw3]
    semgs = [sg0, sg1, sg2, sg3]
    semss = [ss0, ss1, ss2, ss3]
    c = lax.axis_index("c")
    s = lax.axis_index("s")

    def load_unpack_gather(t, idx_p, idx_a, idx_b, rows_v, semg):
        r = s + t * NS

        @pl.when(r < NCH)
        def _():
            pltpu.sync_copy(pk_hbm.at[pl.ds(r * CHUNK, CHUNK)], idx_p)
            for k in range(CHUNK // 16):
                v = idx_p[pl.ds(16 * k, 16)]
                d = v >> PK_SH
                idx_a[pl.ds(16 * k, 16)] = ((v & PK_MASK) + c * N
                                            + (d & 1) * (2 * N))
                idx_b[pl.ds(16 * k, 16)] = d >> 1
            pltpu.async_copy(u_hbm.at[idx_a], rows_v, semg)

        return r

    def scatter(r, idx_a, idx_b, rows_v, semg, sems):
        @pl.when(r < NCH)
        def _():
            pltpu.make_async_copy(u_hbm.at[idx_a], rows_v, semg).wait()
            pltpu.async_copy(rows_v, acc_sp.at[idx_b], sems, add=True)

    def scatter_wait(r, idx_b, rows_v, sems):
        @pl.when(r < NCH)
        def _():
            pltpu.make_async_copy(rows_v, acc_sp.at[idx_b], sems).wait()

    pltpu.sync_copy(zero_hbm, acc_sp.at[pl.ds(s * PSPAN, PSPAN)])
    plsc.subcore_barrier()

    NB = 4
    JJ = (NCH // NS + NB) // NB

    @pl.loop(0, JJ)
    def _loop(j):
        for k in range(NB):
            rp = s + (NB * (j - 1) + k) * NS

            @pl.when((j > 0) & (rp < NCH))
            def _():
                pltpu.make_async_copy(rowss[k], acc_sp.at[idx_bs[k]],
                                      semss[k]).wait()

            load_unpack_gather(NB * j + k, idx_ps[k], idx_as[k], idx_bs[k],
                               rowss[k], semgs[k])
        for k in range(NB):
            r = s + (NB * j + k) * NS
            scatter(r, idx_as[k], idx_bs[k], rowss[k], semgs[k], semss[k])

    for k in range(NB):
        rl = s + (NB * (JJ - 1) + k) * NS
        scatter_wait(rl, idx_bs[k], rowss[k], semss[k])

    plsc.subcore_barrier()
    pltpu.sync_copy(acc_sp.at[pl.ds(s * PSPAN, PSPAN)],
                    out_hbm.at[c * NS + s])


def _segment_sum(packed, zerop, u):
    kfn = pl.kernel(
        _seg_body,
        out_type=jax.ShapeDtypeStruct((NC * NS, PSPAN, 128), jnp.float32),
        mesh=_vector_mesh,
        scratch_types=(
            [pltpu.VMEM((CHUNK,), jnp.int32)] * 12
            + [pltpu.VMEM((CHUNK, 128), jnp.float32)] * 4
            + [pltpu.VMEM_SHARED((PROWS, 128), jnp.float32)]
            + [pltpu.SemaphoreType.DMA] * 8
        ),
    )
    return kfn(packed, zerop, u)


def _mm_body(x_ref, w_ref, y_ref):
    y_ref[...] = jnp.dot(x_ref[0], w_ref[...],
                         preferred_element_type=jnp.float32)


def _project(xt, wcat):
    return pl.pallas_call(
        _mm_body,
        grid=(2,),
        in_specs=[
            pl.BlockSpec((1, N, 128), lambda b: (b, 0, 0)),
            pl.BlockSpec((128, 128), lambda b: (0, 0)),
        ],
        out_specs=pl.BlockSpec((N, 128), lambda b: (b, 0)),
        out_shape=jax.ShapeDtypeStruct((2 * N, 128), jnp.float32),
    )(xt, wcat)


def _scale_body(y_ref, hp_ref, u_ref):
    arr = pl.program_id(0) // 2
    deg = hp_ref[0][:, 0:1] + hp_ref[1][:, 0:1]
    dinv = jnp.where(deg > 0, lax.rsqrt(deg), 0.0)
    u64 = y_ref[:, 64:128] * dinv
    u_ref[:, 0:64] = jnp.where(arr == 0, u64, 0.0)
    u_ref[:, 64:128] = jnp.where(arr == 0, 0.0, u64)


def _scale(y, hp):
    return pl.pallas_call(
        _scale_body,
        grid=(4,),
        in_specs=[
            pl.BlockSpec((N, 128), lambda g: (g % 2, 0)),
            pl.BlockSpec((2, N, 16), lambda g: (0, 0, 0)),
        ],
        out_specs=pl.BlockSpec((N, 128), lambda g: (g, 0)),
        out_shape=jax.ShapeDtypeStruct((4 * N, 128), jnp.float32),
    )(y, hp)


def _comb_body(y_ref, s_ref, hp_ref, bz_ref, bh_ref, wl_ref, bl_ref, o_ref):
    deg = hp_ref[0][:, 0:1] + hp_ref[1][:, 0:1]
    dinv = jnp.where(deg > 0, lax.rsqrt(deg), 0.0)
    a = -dinv * s_ref[...]
    z = jax.nn.sigmoid(y_ref[:, 0:32] + a[:, 0:32] + bz_ref[...])
    ht = jnp.tanh(y_ref[:, 32:64] + a[:, 32:64] + bh_ref[...])
    h = (1.0 - z) * ht
    m = jnp.sum(h, axis=0, keepdims=True) * (1.0 / N)
    b = pl.program_id(0)
    o_ref[pl.ds(b, 1), :] = jax.nn.relu(m) @ wl_ref[...] + bl_ref[...]


def _combine(y, sagg, hp, bz, bh, wlin, blin):
    return pl.pallas_call(
        _comb_body,
        grid=(2,),
        in_specs=[
            pl.BlockSpec((N, 128), lambda b: (b, 0)),
            pl.BlockSpec((N, 64), lambda b: (b, 0)),
            pl.BlockSpec((2, N, 16), lambda b: (0, 0, 0)),
            pl.BlockSpec((1, 32), lambda b: (0, 0)),
            pl.BlockSpec((1, 32), lambda b: (0, 0)),
            pl.BlockSpec((32, 8), lambda b: (0, 0)),
            pl.BlockSpec((1, 8), lambda b: (0, 0)),
        ],
        out_specs=pl.BlockSpec((2, 8), lambda b: (0, 0)),
        out_shape=jax.ShapeDtypeStruct((2, 8), jnp.float32),
    )(y, sagg, hp, bz, bh, wlin, blin)


def _unsplit(o):
    o = o.reshape(NC, 2, RNG, 128)
    return jnp.concatenate([o[:, 0], o[:, 1, :N - RNG]], axis=1)


def _unpair(o):
    o = o.reshape(NC, PROWS, 128)[:, :N // 2]
    s = jnp.stack([o[:, :, 0:64], o[:, :, 64:128]], axis=2)
    return s.reshape(NC * N, 64)


@jax.jit
def _run(x, edge_index, Wxz0, Wxz1, bxz, bhz, Wxh0, Wxh1, bxh, bhh, Wlin,
         blin):
    xt = x[:, -1]
    wcat = jnp.concatenate([Wxz0, Wxh0, Wxz1, Wxh1], axis=1)
    src = edge_index[0]
    dst = edge_index[1]
    packed = (dst << PK_SH) | src
    zero128 = jnp.zeros((ZSPAN, 128), jnp.float32)
    zerop = jnp.zeros((PSPAN, 128), jnp.float32)
    ones128 = jnp.ones((CHUNK, 128), jnp.float32)

    hpo = _degree_partials(packed, zero128, ones128)
    hp = _unsplit(hpo)[:, :, 0:16]
    y = _project(xt, wcat)
    u = _scale(y, hp)
    sagg = _unpair(_segment_sum(packed, zerop, u))
    bz = (bxz + bhz).reshape(1, 32)
    bh = (bxh + bhh).reshape(1, 32)
    return _combine(y, sagg, hp, bz, bh, Wlin, blin.reshape(1, 8))


def kernel(x, edge_index, Wxz0, Wxz1, bxz, Whz0, Whz1, bhz, Wxr0, Wxr1, bxr,
           Whr0, Whr1, bhr, Wxh0, Wxh1, bxh, Whh0, Whh1, bhh, Wlin, blin):
    return _run(x, edge_index, Wxz0, Wxz1, bxz, bhz, Wxh0, Wxh1, bxh, bhh,
                Wlin, blin)


# speedup vs baseline: 24.5227x; 1.0696x over previous
import jax
import jax.numpy as jnp
from jax import lax
from jax.experimental import pallas as pl
from jax.experimental.pallas import tpu as pltpu
from jax.experimental.pallas import tpu_sc as plsc

N = 10000
E = 320000
NC = 2
NS = 16
CHUNK = 128
NCH = E // CHUNK
PK_SH = 14
PK_MASK = (1 << PK_SH) - 1
RNG = 5120
AROWS = 5248
WB = RNG // NS
ZSPAN = AROWS // NS
PROWS = 5008
PSPAN = PROWS // NS

_vector_mesh = plsc.VectorSubcoreMesh(
    core_axis_name="c", subcore_axis_name="s", num_cores=NC, num_subcores=NS)


def _hist_body(pk_hbm, zero_hbm, ones_hbm, out_hbm, ones_v, idx_p0, idx_p1,
               idx_b0, idx_b1, cnt_sp, sem0, sem1):
    c = lax.axis_index("c")
    s = lax.axis_index("s")
    pltpu.sync_copy(ones_hbm, ones_v)
    half = NCH // NC

    def stage(p, t, idx_p, idx_b, sem):
        r = s + t * NS

        @pl.when(r < half)
        def _():
            ch = c * half + r
            pltpu.sync_copy(pk_hbm.at[pl.ds(ch * CHUNK, CHUNK)], idx_p)
            dumpv = RNG + lax.iota(jnp.int32, 16)
            for k in range(CHUNK // 16):
                v = idx_p[pl.ds(16 * k, 16)]
                loc = (v & PK_MASK) - p * RNG
                ok = (loc >= 0) & (loc < RNG)
                idx_b[pl.ds(16 * k, 16)] = jnp.where(ok, loc, dumpv)
            pltpu.async_copy(ones_v, cnt_sp.at[idx_b], sem, add=True)

        return r

    for p in range(2):
        pltpu.sync_copy(zero_hbm, cnt_sp.at[pl.ds(s * ZSPAN, ZSPAN)])
        plsc.subcore_barrier()

        half_jj = (half // NS + 2) // 2

        @pl.loop(0, half_jj)
        def _loop(j):
            rp0 = s + (2 * j - 2) * NS

            @pl.when((j > 0) & (rp0 < half))
            def _():
                pltpu.make_async_copy(ones_v, cnt_sp.at[idx_b0], sem0).wait()

            stage(p, 2 * j, idx_p0, idx_b0, sem0)
            rp1 = s + (2 * j - 1) * NS

            @pl.when((j > 0) & (rp1 < half))
            def _():
                pltpu.make_async_copy(ones_v, cnt_sp.at[idx_b1], sem1).wait()

            stage(p, 2 * j + 1, idx_p1, idx_b1, sem1)

        rl0 = s + (2 * half_jj - 2) * NS
        rl1 = s + (2 * half_jj - 1) * NS

        @pl.when(rl0 < half)
        def _():
            pltpu.make_async_copy(ones_v, cnt_sp.at[idx_b0], sem0).wait()

        @pl.when(rl1 < half)
        def _():
            pltpu.make_async_copy(ones_v, cnt_sp.at[idx_b1], sem1).wait()

        plsc.subcore_barrier()
        pltpu.sync_copy(cnt_sp.at[pl.ds(s * WB, WB)],
                        out_hbm.at[(c * 2 + p) * NS + s])
        plsc.subcore_barrier()


def _degree_partials(packed, zero128, ones128):
    kfn = pl.kernel(
        _hist_body,
        out_type=jax.ShapeDtypeStruct((NC * 2 * NS, WB, 128), jnp.float32),
        mesh=_vector_mesh,
        scratch_types=[
            pltpu.VMEM((CHUNK, 128), jnp.float32),
            pltpu.VMEM((CHUNK,), jnp.int32),
            pltpu.VMEM((CHUNK,), jnp.int32),
            pltpu.VMEM((CHUNK,), jnp.int32),
            pltpu.VMEM((CHUNK,), jnp.int32),
            pltpu.VMEM_SHARED((AROWS, 128), jnp.float32),
            pltpu.SemaphoreType.DMA,
            pltpu.SemaphoreType.DMA,
        ],
    )
    return kfn(packed, zero128, ones128)


def _seg_body(pk_hbm, zero_hbm, u_hbm, out_hbm, ip0, ip1, ip2, ip3, ia0,
              ia1, ia2, ia3, ib0, ib1, ib2, ib3, rw0, rw1, rw2, rw3, acc_sp,
              sg0, sg1, sg2, sg3, ss0, ss1, ss2, ss3):
    idx_ps = [ip0, ip1, ip2, ip3]
    idx_as = [ia0, ia1, ia2, ia3]
    idx_bs = [ib0, ib1, ib2, ib3]
    rowss = [rw0, rw1, rw2, rw3]
    semgs = [sg0, sg1, sg2, sg3]
    semss = [ss0, ss1, ss2, ss3]
    c = lax.axis_index("c")
    s = lax.axis_index("s")

    def load_unpack_gather(t, idx_p, idx_a, idx_b, rows_v, semg):
        r = s + t * NS

        @pl.when(r < NCH)
        def _():
            pltpu.sync_copy(pk_hbm.at[pl.ds(r * CHUNK, CHUNK)], idx_p)
            for k in range(CHUNK // 16):
                v = idx_p[pl.ds(16 * k, 16)]
                d = v >> PK_SH
                idx_a[pl.ds(16 * k, 16)] = ((v & PK_MASK) + c * N
                                            + (d & 1) * (2 * N))
                idx_b[pl.ds(16 * k, 16)] = d >> 1
            pltpu.async_copy(u_hbm.at[idx_a], rows_v, semg)

        return r

    def scatter(r, idx_a, idx_b, rows_v, semg, sems):
        @pl.when(r < NCH)
        def _():
            pltpu.make_async_copy(u_hbm.at[idx_a], rows_v, semg).wait()
            pltpu.async_copy(rows_v, acc_sp.at[idx_b], sems, add=True)

    def scatter_wait(r, idx_b, rows_v, sems):
        @pl.when(r < NCH)
        def _():
            pltpu.make_async_copy(rows_v, acc_sp.at[idx_b], sems).wait()

    pltpu.sync_copy(zero_hbm, acc_sp.at[pl.ds(s * PSPAN, PSPAN)])
    plsc.subcore_barrier()

    NB = 4
    JJ = (NCH // NS + NB) // NB

    @pl.loop(0, JJ)
    def _loop(j):
        for k in range(NB):
            rp = s + (NB * (j - 1) + k) * NS

            @pl.when((j > 0) & (rp < NCH))
            def _():
                pltpu.make_async_copy(rowss[k], acc_sp.at[idx_bs[k]],
                                      semss[k]).wait()

            load_unpack_gather(NB * j + k, idx_ps[k], idx_as[k], idx_bs[k],
                               rowss[k], semgs[k])
        for k in range(NB):
            r = s + (NB * j + k) * NS
            scatter(r, idx_as[k], idx_bs[k], rowss[k], semgs[k], semss[k])

    for k in range(NB):
        rl = s + (NB * (JJ - 1) + k) * NS
        scatter_wait(rl, idx_bs[k], rowss[k], semss[k])

    plsc.subcore_barrier()
    pltpu.sync_copy(acc_sp.at[pl.ds(s * PSPAN, PSPAN)],
                    out_hbm.at[c * NS + s])


def _segment_sum(packed, zerop, u):
    kfn = pl.kernel(
        _seg_body,
        out_type=jax.ShapeDtypeStruct((NC * NS, PSPAN, 128), jnp.float32),
        mesh=_vector_mesh,
        scratch_types=(
            [pltpu.VMEM((CHUNK,), jnp.int32)] * 12
            + [pltpu.VMEM((CHUNK, 128), jnp.float32)] * 4
            + [pltpu.VMEM_SHARED((PROWS, 128), jnp.float32)]
            + [pltpu.SemaphoreType.DMA] * 8
        ),
    )
    return kfn(packed, zerop, u)


def _mm_body(x_ref, w_ref, y_ref):
    y_ref[...] = jnp.dot(x_ref[0], w_ref[...],
                         preferred_element_type=jnp.float32)


def _project(xt, wcat):
    return pl.pallas_call(
        _mm_body,
        grid=(2,),
        in_specs=[
            pl.BlockSpec((1, N, 128), lambda b: (b, 0, 0)),
            pl.BlockSpec((128, 128), lambda b: (0, 0)),
        ],
        out_specs=pl.BlockSpec((N, 128), lambda b: (b, 0)),
        out_shape=jax.ShapeDtypeStruct((2 * N, 128), jnp.float32),
    )(xt, wcat)


def _scale_body(y_ref, hp_ref, u_ref):
    arr = pl.program_id(0) // 2
    deg = hp_ref[0][:, 0:1] + hp_ref[1][:, 0:1]
    dinv = jnp.where(deg > 0, lax.rsqrt(deg), 0.0)
    u64 = y_ref[:, 64:128] * dinv
    u_ref[:, 0:64] = jnp.where(arr == 0, u64, 0.0)
    u_ref[:, 64:128] = jnp.where(arr == 0, 0.0, u64)


def _scale(y, hp):
    return pl.pallas_call(
        _scale_body,
        grid=(4,),
        in_specs=[
            pl.BlockSpec((N, 128), lambda g: (g % 2, 0)),
            pl.BlockSpec((2, N, 16), lambda g: (0, 0, 0)),
        ],
        out_specs=pl.BlockSpec((N, 128), lambda g: (g, 0)),
        out_shape=jax.ShapeDtypeStruct((4 * N, 128), jnp.float32),
    )(y, hp)


def _comb_body(y_ref, s_ref, hp_ref, bz_ref, bh_ref, wl_ref, bl_ref, o_ref):
    deg = hp_ref[0][:, 0:1] + hp_ref[1][:, 0:1]
    dinv = jnp.where(deg > 0, lax.rsqrt(deg), 0.0)
    a = -dinv * s_ref[...]
    z = jax.nn.sigmoid(y_ref[:, 0:32] + a[:, 0:32] + bz_ref[...])
    ht = jnp.tanh(y_ref[:, 32:64] + a[:, 32:64] + bh_ref[...])
    h = (1.0 - z) * ht
    m = jnp.sum(h, axis=0, keepdims=True) * (1.0 / N)
    b = pl.program_id(0)
    o_ref[pl.ds(b, 1), :] = jax.nn.relu(m) @ wl_ref[...] + bl_ref[...]


def _combine(y, sagg, hp, bz, bh, wlin, blin):
    return pl.pallas_call(
        _comb_body,
        grid=(2,),
        in_specs=[
            pl.BlockSpec((N, 128), lambda b: (b, 0)),
            pl.BlockSpec((N, 64), lambda b: (b, 0)),
            pl.BlockSpec((2, N, 16), lambda b: (0, 0, 0)),
            pl.BlockSpec((1, 32), lambda b: (0, 0)),
            pl.BlockSpec((1, 32), lambda b: (0, 0)),
            pl.BlockSpec((32, 8), lambda b: (0, 0)),
            pl.BlockSpec((1, 8), lambda b: (0, 0)),
        ],
        out_specs=pl.BlockSpec((2, 8), lambda b: (0, 0)),
        out_shape=jax.ShapeDtypeStruct((2, 8), jnp.float32),
    )(y, sagg, hp, bz, bh, wlin, blin)


def _unsplit(o):
    o = o.reshape(NC, 2, RNG, 128)
    return jnp.concatenate([o[:, 0], o[:, 1, :N - RNG]], axis=1)


def _unpair(o):
    o = o.reshape(NC, PROWS, 128)[:, :N // 2]
    s = jnp.stack([o[:, :, 0:64], o[:, :, 64:128]], axis=2)
    return s.reshape(NC * N, 64)


@jax.jit
def _run(x, edge_index, Wxz0, Wxz1, bxz, bhz, Wxh0, Wxh1, bxh, bhh, Wlin,
         blin):
    xt = x[:, -1]
    wcat = jnp.concatenate([Wxz0, Wxh0, Wxz1, Wxh1], axis=1)
    src = edge_index[0]
    dst = edge_index[1]
    packed = (dst << PK_SH) | src
    zero128 = jnp.zeros((ZSPAN, 128), jnp.float32)
    zerop = jnp.zeros((PSPAN, 128), jnp.float32)
    ones128 = jnp.ones((CHUNK, 128), jnp.float32)

    hpo = _degree_partials(packed, zero128, ones128)
    hp = _unsplit(hpo)[:, :, 0:16]
    y = _project(xt, wcat)
    u = _scale(y, hp)
    sagg = _unpair(_segment_sum(packed, zerop, u))
    bz = (bxz + bhz).reshape(1, 32)
    bh = (bxh + bhh).reshape(1, 32)
    return _combine(y, sagg, hp, bz, bh, Wlin, blin.reshape(1, 8))


def kernel(x, edge_index, Wxz0, Wxz1, bxz, Whz0, Whz1, bhz, Wxr0, Wxr1, bxr,
           Whr0, Whr1, bhr, Wxh0, Wxh1, bxh, Whh0, Whh1, bhh, Wlin, blin):
    return _run(x, edge_index, Wxz0, Wxz1, bxz, bhz, Wxh0, Wxh1, bxh, bhh,
                Wlin, blin)
